# baseline (device time: 67923 ns/iter reference)
import numpy as np

import jax
import jax.numpy as jnp
from jax import lax
from jax.experimental import pallas as pl
from jax.experimental.pallas import tpu as pltpu

N_DEV = 16
N_HEADS = 8
N_PAIR = N_HEADS // 2
DH = 128
SQ = 512
SKV_LOCAL = 2048
D = N_HEADS * DH
SEG = D // N_DEV
ROWS = SEG // N_HEADS
BLK = D // 4
SCALE = 0.08838834764831843
LOG2E = 1.4426950408889634
SCALE2 = SCALE * LOG2E

_PERM = np.array([h * DH + d * ROWS + k
                  for d in range(N_DEV)
                  for h in range(N_HEADS)
                  for k in range(ROWS)])


def kernel(x, Wq, Wo, K_ext, V_ext):
    xb = x.reshape(SQ, D).astype(jnp.bfloat16)
    Wqb = Wq.astype(jnp.bfloat16)
    Wob = Wo[_PERM, :].astype(jnp.bfloat16)
    Kb = K_ext.reshape(SKV_LOCAL, D).astype(jnp.bfloat16)
    Vb = V_ext.reshape(SKV_LOCAL, D).astype(jnp.bfloat16)

    def body(x_ref, wq_ref, wo_ref, k_ref, v_ref, out_ref,
             catbuf, statbuf, inbox, statsin,
             a_send, a_recv, b_send, b_recv, d_send, d_recv, exit_sems):
        my = lax.axis_index("i")

        barrier = pltpu.get_barrier_semaphore()
        for k in range(1, N_DEV):
            pl.semaphore_signal(barrier, inc=1,
                                device_id=(lax.rem(my + k, N_DEV),),
                                device_id_type=pl.DeviceIdType.MESH)
        pl.semaphore_wait(barrier, N_DEV - 1)

        x2 = x_ref[...]
        for h in range(N_HEADS):
            q = jnp.dot(x2, wq_ref[:, h * DH:(h + 1) * DH],
                        preferred_element_type=jnp.float32)
            q = (q * SCALE2).astype(jnp.bfloat16)
            kh = k_ref[:, h * DH:(h + 1) * DH]
            vh = v_ref[:, h * DH:(h + 1) * DH]
            st = lax.dot_general(kh, q, (((1,), (1,)), ((), ())),
                                 preferred_element_type=jnp.float32)
            m = jnp.max(st, axis=0, keepdims=True)
            m = m.astype(jnp.bfloat16).astype(jnp.float32)
            p = jnp.exp2(st - m)
            l = jnp.sum(p, axis=0, keepdims=True)
            ot = lax.dot_general(vh, p.astype(jnp.bfloat16),
                                 (((0,), (0,)), ((), ())),
                                 preferred_element_type=jnp.float32)
            otb = ot.astype(jnp.bfloat16)
            for d in range(N_DEV):
                catbuf[d * SEG + h * ROWS:d * SEG + h * ROWS + ROWS, :] = (
                    otb[d * ROWS:(d + 1) * ROWS, :])
            statbuf[h:h + 1, :] = m.astype(jnp.bfloat16)
            statbuf[N_HEADS + h:N_HEADS + h + 1, :] = l.astype(jnp.bfloat16)

            if h % 2 == 1:
                g = h // 2
                for peer in range(N_DEV):
                    @pl.when(peer != my)
                    def _(g=g, peer=peer):
                        pltpu.make_async_remote_copy(
                            src_ref=catbuf.at[
                                pl.ds(peer * SEG + g * 2 * ROWS, 2 * ROWS), :],
                            dst_ref=inbox.at[my, g],
                            send_sem=b_send.at[peer * N_PAIR + g],
                            recv_sem=b_recv.at[my * N_PAIR + g],
                            device_id=(peer,),
                            device_id_type=pl.DeviceIdType.MESH).start()

                    @pl.when(peer == my)
                    def _(g=g):
                        inbox[pl.ds(my, 1), g] = catbuf[
                            pl.ds(my * SEG + g * 2 * ROWS, 2 * ROWS), :][None]

        for peer in range(N_DEV):
            @pl.when(peer != my)
            def _(peer=peer):
                pltpu.make_async_remote_copy(
                    src_ref=statbuf, dst_ref=statsin.at[my],
                    send_sem=a_send.at[peer], recv_sem=a_recv.at[my],
                    device_id=(peer,),
                    device_id_type=pl.DeviceIdType.MESH).start()

            @pl.when(peer == my)
            def _():
                statsin[pl.ds(my, 1)] = statbuf[...][None]

        for s in range(N_DEV):
            @pl.when(s != my)
            def _(s=s):
                for g in range(N_PAIR):
                    pltpu.make_async_remote_copy(
                        src_ref=inbox.at[s, g], dst_ref=inbox.at[s, g],
                        send_sem=b_send.at[s * N_PAIR + g],
                        recv_sem=b_recv.at[s * N_PAIR + g],
                        device_id=(my,),
                        device_id_type=pl.DeviceIdType.MESH).wait_recv()
                pltpu.make_async_remote_copy(
                    src_ref=statsin.at[s], dst_ref=statsin.at[s],
                    send_sem=a_send.at[s], recv_sem=a_recv.at[s],
                    device_id=(my,),
                    device_id_type=pl.DeviceIdType.MESH).wait_recv()

        parts = []
        for h in range(N_HEADS):
            g, half = h // 2, (h % 2) * ROWS
            ms = [statsin[s, h:h + 1, :].astype(jnp.float32)
                  for s in range(N_DEV)]
            ls = [statsin[s, N_HEADS + h:N_HEADS + h + 1, :].astype(jnp.float32)
                  for s in range(N_DEV)]
            mg = ms[0]
            for s in range(1, N_DEV):
                mg = jnp.maximum(mg, ms[s])
            num = jnp.zeros((ROWS, SQ), jnp.float32)
            den = jnp.zeros((1, SQ), jnp.float32)
            for s in range(N_DEV):
                w = jnp.exp2(ms[s] - mg)
                num = num + inbox[s, g, half:half + ROWS, :].astype(
                    jnp.float32) * w
                den = den + ls[s] * w
            parts.append(num / den)
        seg_norm = jnp.concatenate(parts, axis=0).astype(jnp.bfloat16)
        catbuf[pl.ds(my * SEG, SEG), :] = seg_norm

        for peer in range(N_DEV):
            @pl.when(peer != my)
            def _(peer=peer):
                pltpu.make_async_remote_copy(
                    src_ref=catbuf.at[pl.ds(my * SEG, SEG), :],
                    dst_ref=catbuf.at[pl.ds(my * SEG, SEG), :],
                    send_sem=d_send.at[peer], recv_sem=d_recv.at[my],
                    device_id=(peer,),
                    device_id_type=pl.DeviceIdType.MESH).start()

        final = jnp.zeros((SQ, D), jnp.float32)
        for g in range(4):
            for s in range(4 * g, 4 * g + 4):
                @pl.when(s != my)
                def _(s=s):
                    pltpu.make_async_remote_copy(
                        src_ref=catbuf.at[pl.ds(s * SEG, SEG), :],
                        dst_ref=catbuf.at[pl.ds(s * SEG, SEG), :],
                        send_sem=d_send.at[s], recv_sem=d_recv.at[s],
                        device_id=(my,),
                        device_id_type=pl.DeviceIdType.MESH,
                    ).wait_recv()
            final = final + lax.dot_general(
                catbuf[pl.ds(g * BLK, BLK), :],
                wo_ref[pl.ds(g * BLK, BLK), :],
                (((0,), (0,)), ((), ())),
                preferred_element_type=jnp.float32)
        out_ref[...] = final

        for peer in range(N_DEV):
            @pl.when(peer != my)
            def _(peer=peer):
                for g in range(N_PAIR):
                    pltpu.make_async_remote_copy(
                        src_ref=inbox.at[peer, g], dst_ref=inbox.at[peer, g],
                        send_sem=b_send.at[peer * N_PAIR + g],
                        recv_sem=b_recv.at[peer * N_PAIR + g],
                        device_id=(peer,),
                        device_id_type=pl.DeviceIdType.MESH).wait_send()
                pltpu.make_async_remote_copy(
                    src_ref=statsin.at[peer], dst_ref=statsin.at[peer],
                    send_sem=a_send.at[peer], recv_sem=a_recv.at[peer],
                    device_id=(peer,), device_id_type=pl.DeviceIdType.MESH,
                ).wait_send()
                pltpu.make_async_remote_copy(
                    src_ref=catbuf.at[pl.ds(peer * SEG, SEG), :],
                    dst_ref=catbuf.at[pl.ds(peer * SEG, SEG), :],
                    send_sem=d_send.at[peer], recv_sem=d_recv.at[peer],
                    device_id=(peer,), device_id_type=pl.DeviceIdType.MESH,
                ).wait_send()

        for k in range(1, N_DEV):
            pl.semaphore_signal(exit_sems.at[N_DEV - k - 1], inc=1,
                                device_id=(lax.rem(my + k, N_DEV),),
                                device_id_type=pl.DeviceIdType.MESH)
        for j in range(1, N_DEV):
            pl.semaphore_wait(exit_sems.at[j - 1], 1)

    out = pl.pallas_call(
        body,
        out_shape=jax.ShapeDtypeStruct((SQ, D), jnp.float32),
        in_specs=[pl.BlockSpec(memory_space=pltpu.VMEM)] * 5,
        out_specs=pl.BlockSpec(memory_space=pltpu.VMEM),
        scratch_shapes=[
            pltpu.VMEM((D, SQ), jnp.bfloat16),
            pltpu.VMEM((2 * N_HEADS, SQ), jnp.bfloat16),
            pltpu.VMEM((N_DEV, N_PAIR, 2 * ROWS, SQ), jnp.bfloat16),
            pltpu.VMEM((N_DEV, 2 * N_HEADS, SQ), jnp.bfloat16),
            pltpu.SemaphoreType.DMA((N_DEV,)),
            pltpu.SemaphoreType.DMA((N_DEV,)),
            pltpu.SemaphoreType.DMA((N_DEV * N_PAIR,)),
            pltpu.SemaphoreType.DMA((N_DEV * N_PAIR,)),
            pltpu.SemaphoreType.DMA((N_DEV,)),
            pltpu.SemaphoreType.DMA((N_DEV,)),
            pltpu.SemaphoreType.REGULAR((N_DEV - 1,)),
        ],
        compiler_params=pltpu.CompilerParams(collective_id=0),
    )(xb, Wqb, Wob, Kb, Vb)
    return out.reshape(1, SQ, D)


# device time: 65321 ns/iter; 1.0398x vs baseline; 1.0398x over previous
import jax
import jax.numpy as jnp
from jax import lax
from jax.experimental import pallas as pl
from jax.experimental.pallas import tpu as pltpu

N_DEV = 16
N_HEADS = 8
DH = 128
SQ = 512
SKV_LOCAL = 2048
D = N_HEADS * DH
SEG = D // N_DEV
BLK = D // 4
SCALE = 0.08838834764831843
LOG2E = 1.4426950408889634
SCALE2 = SCALE * LOG2E


def kernel(x, Wq, Wo, K_ext, V_ext):
    xb = x.reshape(SQ, D).astype(jnp.bfloat16)
    Wqb = Wq.astype(jnp.bfloat16)
    Wob = Wo.astype(jnp.bfloat16)
    Kb = K_ext.reshape(SKV_LOCAL, D).astype(jnp.bfloat16)
    Vb = V_ext.reshape(SKV_LOCAL, D).astype(jnp.bfloat16)

    def body(x_ref, wq_ref, wo_ref, k_ref, v_ref, out_ref,
             catbuf, statbuf, oseg_all, stats_all,
             a_send, a_recv, b_send, b_recv, d_send, d_recv, exit_sems):
        my = lax.axis_index("i")

        barrier = pltpu.get_barrier_semaphore()
        for k in range(1, N_DEV):
            pl.semaphore_signal(barrier, inc=1,
                                device_id=(lax.rem(my + k, N_DEV),),
                                device_id_type=pl.DeviceIdType.MESH)
        pl.semaphore_wait(barrier, N_DEV - 1)

        x2 = x_ref[...]
        for h in range(N_HEADS):
            q = jnp.dot(x2, wq_ref[:, h * DH:(h + 1) * DH],
                        preferred_element_type=jnp.float32)
            q = (q * SCALE2).astype(jnp.bfloat16)
            kh = k_ref[:, h * DH:(h + 1) * DH]
            vh = v_ref[:, h * DH:(h + 1) * DH]
            st = lax.dot_general(kh, q, (((1,), (1,)), ((), ())),
                                 preferred_element_type=jnp.float32)
            m = jnp.max(st, axis=0, keepdims=True)
            p = jnp.exp2(st - m)
            l = jnp.sum(p, axis=0, keepdims=True)
            ot = lax.dot_general(vh, p.astype(jnp.bfloat16),
                                 (((0,), (0,)), ((), ())),
                                 preferred_element_type=jnp.float32)
            catbuf[h * DH:(h + 1) * DH, :] = ot.astype(jnp.bfloat16)
            statbuf[h, 0:1, :] = m
            statbuf[h, 1:2, :] = l
            stat = jnp.concatenate([m, l], axis=0)

            for peer in (2 * h, 2 * h + 1):
                seg = ot[(peer % 2) * SEG:(peer % 2) * SEG + SEG, :]
                seg = seg.astype(jnp.bfloat16)

                @pl.when(peer == my)
                def _(seg=seg, stat=stat):
                    oseg_all[pl.ds(my, 1)] = seg[None]
                    stats_all[pl.ds(my, 1)] = stat[None]

                @pl.when(peer != my)
                def _(h=h, peer=peer):
                    pltpu.make_async_remote_copy(
                        src_ref=statbuf.at[h],
                        dst_ref=stats_all.at[my],
                        send_sem=a_send.at[peer], recv_sem=a_recv.at[my],
                        device_id=(peer,),
                        device_id_type=pl.DeviceIdType.MESH).start()
                    pltpu.make_async_remote_copy(
                        src_ref=catbuf.at[pl.ds(peer * SEG, SEG), :],
                        dst_ref=oseg_all.at[my],
                        send_sem=b_send.at[peer], recv_sem=b_recv.at[my],
                        device_id=(peer,),
                        device_id_type=pl.DeviceIdType.MESH).start()

        for s in range(N_DEV):
            @pl.when(s != my)
            def _(s=s):
                pltpu.make_async_remote_copy(
                    src_ref=stats_all.at[s], dst_ref=stats_all.at[s],
                    send_sem=a_send.at[s], recv_sem=a_recv.at[s],
                    device_id=(my,), device_id_type=pl.DeviceIdType.MESH,
                ).wait_recv()
                pltpu.make_async_remote_copy(
                    src_ref=oseg_all.at[s], dst_ref=oseg_all.at[s],
                    send_sem=b_send.at[s], recv_sem=b_recv.at[s],
                    device_id=(my,), device_id_type=pl.DeviceIdType.MESH,
                ).wait_recv()

        ms = [stats_all[s, 0:1, :] for s in range(N_DEV)]
        ls = [stats_all[s, 1:2, :] for s in range(N_DEV)]
        mg = ms[0]
        for s in range(1, N_DEV):
            mg = jnp.maximum(mg, ms[s])
        num = jnp.zeros((SEG, SQ), jnp.float32)
        den = jnp.zeros((1, SQ), jnp.float32)
        for s in range(N_DEV):
            w = jnp.exp2(ms[s] - mg)
            num = num + oseg_all[s].astype(jnp.float32) * w
            den = den + ls[s] * w
        seg_norm = (num / den).astype(jnp.bfloat16)
        catbuf[pl.ds(my * SEG, SEG), :] = seg_norm

        for peer in range(N_DEV):
            @pl.when(peer != my)
            def _(peer=peer):
                pltpu.make_async_remote_copy(
                    src_ref=catbuf.at[pl.ds(my * SEG, SEG), :],
                    dst_ref=catbuf.at[pl.ds(my * SEG, SEG), :],
                    send_sem=d_send.at[peer], recv_sem=d_recv.at[my],
                    device_id=(peer,),
                    device_id_type=pl.DeviceIdType.MESH).start()

        final = jnp.zeros((SQ, D), jnp.float32)
        for g in range(4):
            for s in range(4 * g, 4 * g + 4):
                @pl.when(s != my)
                def _(s=s):
                    pltpu.make_async_remote_copy(
                        src_ref=catbuf.at[pl.ds(s * SEG, SEG), :],
                        dst_ref=catbuf.at[pl.ds(s * SEG, SEG), :],
                        send_sem=d_send.at[s], recv_sem=d_recv.at[s],
                        device_id=(my,),
                        device_id_type=pl.DeviceIdType.MESH,
                    ).wait_recv()
            final = final + lax.dot_general(
                catbuf[pl.ds(g * BLK, BLK), :],
                wo_ref[pl.ds(g * BLK, BLK), :],
                (((0,), (0,)), ((), ())),
                preferred_element_type=jnp.float32)
        out_ref[0] = final

        for peer in range(N_DEV):
            @pl.when(peer != my)
            def _(peer=peer):
                pltpu.make_async_remote_copy(
                    src_ref=stats_all.at[peer], dst_ref=stats_all.at[peer],
                    send_sem=a_send.at[peer], recv_sem=a_recv.at[peer],
                    device_id=(peer,), device_id_type=pl.DeviceIdType.MESH,
                ).wait_send()
                pltpu.make_async_remote_copy(
                    src_ref=oseg_all.at[peer], dst_ref=oseg_all.at[peer],
                    send_sem=b_send.at[peer], recv_sem=b_recv.at[peer],
                    device_id=(peer,), device_id_type=pl.DeviceIdType.MESH,
                ).wait_send()
                pltpu.make_async_remote_copy(
                    src_ref=catbuf.at[pl.ds(peer * SEG, SEG), :],
                    dst_ref=catbuf.at[pl.ds(peer * SEG, SEG), :],
                    send_sem=d_send.at[peer], recv_sem=d_recv.at[peer],
                    device_id=(peer,), device_id_type=pl.DeviceIdType.MESH,
                ).wait_send()

        for k in range(1, N_DEV):
            pl.semaphore_signal(exit_sems.at[N_DEV - k - 1], inc=1,
                                device_id=(lax.rem(my + k, N_DEV),),
                                device_id_type=pl.DeviceIdType.MESH)
        for j in range(1, N_DEV):
            pl.semaphore_wait(exit_sems.at[j - 1], 1)

    out = pl.pallas_call(
        body,
        out_shape=jax.ShapeDtypeStruct((1, SQ, D), jnp.float32),
        in_specs=[pl.BlockSpec(memory_space=pltpu.VMEM)] * 5,
        out_specs=pl.BlockSpec(memory_space=pltpu.VMEM),
        scratch_shapes=[
            pltpu.VMEM((D, SQ), jnp.bfloat16),
            pltpu.VMEM((N_HEADS, 2, SQ), jnp.float32),
            pltpu.VMEM((N_DEV, SEG, SQ), jnp.bfloat16),
            pltpu.VMEM((N_DEV, 2, SQ), jnp.float32),
            pltpu.SemaphoreType.DMA((N_DEV,)),
            pltpu.SemaphoreType.DMA((N_DEV,)),
            pltpu.SemaphoreType.DMA((N_DEV,)),
            pltpu.SemaphoreType.DMA((N_DEV,)),
            pltpu.SemaphoreType.DMA((N_DEV,)),
            pltpu.SemaphoreType.DMA((N_DEV,)),
            pltpu.SemaphoreType.REGULAR((N_DEV - 1,)),
        ],
        compiler_params=pltpu.CompilerParams(collective_id=0),
    )(xb, Wqb, Wob, Kb, Vb)
    return out


# device time: 63681 ns/iter; 1.0666x vs baseline; 1.0258x over previous
import jax
import jax.numpy as jnp
from jax import lax
from jax.experimental import pallas as pl
from jax.experimental.pallas import tpu as pltpu

N_DEV = 16
N_HEADS = 8
DH = 128
SQ = 512
SKV_LOCAL = 2048
D = N_HEADS * DH
SEG = D // N_DEV
BLK = D // 4
SCALE = 0.08838834764831843
LOG2E = 1.4426950408889634
SCALE2 = SCALE * LOG2E


def kernel(x, Wq, Wo, K_ext, V_ext):
    xb = x.reshape(SQ, D).astype(jnp.bfloat16)
    Wqb = Wq.astype(jnp.bfloat16)
    Wob = Wo.astype(jnp.bfloat16)
    Kb = K_ext.reshape(SKV_LOCAL, D).astype(jnp.bfloat16)
    Vb = V_ext.reshape(SKV_LOCAL, D).astype(jnp.bfloat16)

    def body(x_ref, wq_ref, wo_ref, k_ref, v_ref, out_ref,
             catbuf, statbuf, oseg_all, stats_all,
             a_send, a_recv, b_send, b_recv, d_send, d_recv, exit_sems):
        my = lax.axis_index("i")

        barrier = pltpu.get_barrier_semaphore()
        for k in range(1, N_DEV):
            pl.semaphore_signal(barrier, inc=1,
                                device_id=(lax.rem(my + k, N_DEV),),
                                device_id_type=pl.DeviceIdType.MESH)

        x2 = x_ref[...]
        q_all = jnp.dot(x2, wq_ref[...],
                        preferred_element_type=jnp.float32)
        q_all = (q_all * SCALE2).astype(jnp.bfloat16)
        for h in range(N_HEADS):
            q = q_all[:, h * DH:(h + 1) * DH]
            kh = k_ref[:, h * DH:(h + 1) * DH]
            vh = v_ref[:, h * DH:(h + 1) * DH]
            st = lax.dot_general(kh, q, (((1,), (1,)), ((), ())),
                                 preferred_element_type=jnp.float32)
            m = jnp.max(st, axis=0, keepdims=True)
            p = jnp.exp2(st - m)
            l = jnp.sum(p, axis=0, keepdims=True)
            ot = lax.dot_general(vh, p.astype(jnp.bfloat16),
                                 (((0,), (0,)), ((), ())),
                                 preferred_element_type=jnp.float32)
            catbuf[h * DH:(h + 1) * DH, :] = ot.astype(jnp.bfloat16)
            statbuf[h, 0:1, :] = m
            statbuf[h, 1:2, :] = l
            stat = jnp.concatenate([m, l], axis=0)

            if h == 0:
                pl.semaphore_wait(barrier, N_DEV - 1)

            for peer in (2 * h, 2 * h + 1):
                seg = ot[(peer % 2) * SEG:(peer % 2) * SEG + SEG, :]
                seg = seg.astype(jnp.bfloat16)

                @pl.when(peer == my)
                def _(seg=seg, stat=stat):
                    oseg_all[pl.ds(my, 1)] = seg[None]
                    stats_all[pl.ds(my, 1)] = stat[None]

                @pl.when(peer != my)
                def _(h=h, peer=peer):
                    pltpu.make_async_remote_copy(
                        src_ref=statbuf.at[h],
                        dst_ref=stats_all.at[my],
                        send_sem=a_send.at[peer], recv_sem=a_recv.at[my],
                        device_id=(peer,),
                        device_id_type=pl.DeviceIdType.MESH).start()
                    pltpu.make_async_remote_copy(
                        src_ref=catbuf.at[pl.ds(peer * SEG, SEG), :],
                        dst_ref=oseg_all.at[my],
                        send_sem=b_send.at[peer], recv_sem=b_recv.at[my],
                        device_id=(peer,),
                        device_id_type=pl.DeviceIdType.MESH).start()

        for s in range(N_DEV):
            @pl.when(s != my)
            def _(s=s):
                pltpu.make_async_remote_copy(
                    src_ref=stats_all.at[s], dst_ref=stats_all.at[s],
                    send_sem=a_send.at[s], recv_sem=a_recv.at[s],
                    device_id=(my,), device_id_type=pl.DeviceIdType.MESH,
                ).wait_recv()
                pltpu.make_async_remote_copy(
                    src_ref=oseg_all.at[s], dst_ref=oseg_all.at[s],
                    send_sem=b_send.at[s], recv_sem=b_recv.at[s],
                    device_id=(my,), device_id_type=pl.DeviceIdType.MESH,
                ).wait_recv()

        ms = [stats_all[s, 0:1, :] for s in range(N_DEV)]
        ls = [stats_all[s, 1:2, :] for s in range(N_DEV)]
        mg = ms[0]
        for s in range(1, N_DEV):
            mg = jnp.maximum(mg, ms[s])
        num = jnp.zeros((SEG, SQ), jnp.float32)
        den = jnp.zeros((1, SQ), jnp.float32)
        for s in range(N_DEV):
            w = jnp.exp2(ms[s] - mg)
            num = num + oseg_all[s].astype(jnp.float32) * w
            den = den + ls[s] * w
        seg_norm = (num / den).astype(jnp.bfloat16)
        catbuf[pl.ds(my * SEG, SEG), :] = seg_norm

        for peer in range(N_DEV):
            @pl.when(peer != my)
            def _(peer=peer):
                pltpu.make_async_remote_copy(
                    src_ref=catbuf.at[pl.ds(my * SEG, SEG), :],
                    dst_ref=catbuf.at[pl.ds(my * SEG, SEG), :],
                    send_sem=d_send.at[peer], recv_sem=d_recv.at[my],
                    device_id=(peer,),
                    device_id_type=pl.DeviceIdType.MESH).start()

        final = jnp.zeros((SQ, D), jnp.float32)
        for g in range(4):
            for s in range(4 * g, 4 * g + 4):
                @pl.when(s != my)
                def _(s=s):
                    pltpu.make_async_remote_copy(
                        src_ref=catbuf.at[pl.ds(s * SEG, SEG), :],
                        dst_ref=catbuf.at[pl.ds(s * SEG, SEG), :],
                        send_sem=d_send.at[s], recv_sem=d_recv.at[s],
                        device_id=(my,),
                        device_id_type=pl.DeviceIdType.MESH,
                    ).wait_recv()
            final = final + lax.dot_general(
                catbuf[pl.ds(g * BLK, BLK), :],
                wo_ref[pl.ds(g * BLK, BLK), :],
                (((0,), (0,)), ((), ())),
                preferred_element_type=jnp.float32)
        out_ref[0] = final

        for peer in range(N_DEV):
            @pl.when(peer != my)
            def _(peer=peer):
                pltpu.make_async_remote_copy(
                    src_ref=stats_all.at[peer], dst_ref=stats_all.at[peer],
                    send_sem=a_send.at[peer], recv_sem=a_recv.at[peer],
                    device_id=(peer,), device_id_type=pl.DeviceIdType.MESH,
                ).wait_send()
                pltpu.make_async_remote_copy(
                    src_ref=oseg_all.at[peer], dst_ref=oseg_all.at[peer],
                    send_sem=b_send.at[peer], recv_sem=b_recv.at[peer],
                    device_id=(peer,), device_id_type=pl.DeviceIdType.MESH,
                ).wait_send()
                pltpu.make_async_remote_copy(
                    src_ref=catbuf.at[pl.ds(peer * SEG, SEG), :],
                    dst_ref=catbuf.at[pl.ds(peer * SEG, SEG), :],
                    send_sem=d_send.at[peer], recv_sem=d_recv.at[peer],
                    device_id=(peer,), device_id_type=pl.DeviceIdType.MESH,
                ).wait_send()

        for k in range(1, N_DEV):
            pl.semaphore_signal(exit_sems.at[N_DEV - k - 1], inc=1,
                                device_id=(lax.rem(my + k, N_DEV),),
                                device_id_type=pl.DeviceIdType.MESH)
        for j in range(1, N_DEV):
            pl.semaphore_wait(exit_sems.at[j - 1], 1)

    out = pl.pallas_call(
        body,
        out_shape=jax.ShapeDtypeStruct((1, SQ, D), jnp.float32),
        in_specs=[pl.BlockSpec(memory_space=pltpu.VMEM)] * 5,
        out_specs=pl.BlockSpec(memory_space=pltpu.VMEM),
        scratch_shapes=[
            pltpu.VMEM((D, SQ), jnp.bfloat16),
            pltpu.VMEM((N_HEADS, 2, SQ), jnp.float32),
            pltpu.VMEM((N_DEV, SEG, SQ), jnp.bfloat16),
            pltpu.VMEM((N_DEV, 2, SQ), jnp.float32),
            pltpu.SemaphoreType.DMA((N_DEV,)),
            pltpu.SemaphoreType.DMA((N_DEV,)),
            pltpu.SemaphoreType.DMA((N_DEV,)),
            pltpu.SemaphoreType.DMA((N_DEV,)),
            pltpu.SemaphoreType.DMA((N_DEV,)),
            pltpu.SemaphoreType.DMA((N_DEV,)),
            pltpu.SemaphoreType.REGULAR((N_DEV - 1,)),
        ],
        compiler_params=pltpu.CompilerParams(collective_id=0),
    )(xb, Wqb, Wob, Kb, Vb)
    return out
